# Initial kernel scaffold; baseline (speedup 1.0000x reference)
import jax
import jax.numpy as jnp
from jax.experimental import pallas as pl


def kernel(nodes, edges, senders, receivers, copy_arr, Wq, bq, W1, b1, ln1_s, ln1_b, W2, b2, ln2_s, ln2_b, W3, b3):
    raise NotImplementedError("write your pallas kernel here")


# speedup vs baseline: 5.4084x; 5.4084x over previous
import functools

import jax
import jax.numpy as jnp
from jax import lax
from jax.experimental import pallas as pl
from jax.experimental.pallas import tpu as pltpu
from jax.experimental.pallas import tpu_sc as plsc

N = 10000
E = 320000
D = 128
DE = 16
F0 = 128
F1 = 64

E_SELF = E + N
E_PAD = 331776
N_PAD = 10240

NC = 2
NS = 16
NW = NC * NS

DP = D // 2
G_ROWS = 2 * E_PAD
G_RPW = G_ROWS // NW
G_CH = 432
G_NCH = G_RPW // G_CH
G_TILE_N = N_PAD // NS


def _gather_body(nodes_hbm, idx_hbm, out_hbm, tbl_sh, idx_v, rows0, rows1,
                 gsem, wsem0, wsem1):
    c = lax.axis_index("c")
    s = lax.axis_index("s")
    wid = s * NC + c
    base_w = wid * G_RPW
    pltpu.sync_copy(nodes_hbm.at[pl.ds(s * G_TILE_N, G_TILE_N)],
                    tbl_sh.at[pl.ds(s * G_TILE_N, G_TILE_N)])
    pltpu.sync_copy(idx_hbm.at[pl.ds(base_w, G_RPW)], idx_v)
    plsc.subcore_barrier()

    bufs = (rows0, rows1)
    sems = (wsem0, wsem1)

    def body2(j2, _):
        for b in range(2):
            j = 2 * j2 + b
            buf, sem = bufs[b], sems[b]

            @pl.when(j2 >= 1)
            def _():
                pltpu.make_async_copy(
                    buf, out_hbm.at[pl.ds(0, G_CH)], sem).wait()

            pltpu.async_copy(
                tbl_sh.at[idx_v.at[pl.ds(j * G_CH, G_CH)]], buf,
                gsem).wait()
            pltpu.make_async_copy(
                buf, out_hbm.at[pl.ds(base_w + j * G_CH, G_CH)], sem
            ).start()
        return 0

    lax.fori_loop(0, G_NCH // 2, body2, 0)
    for b in range(2):
        pltpu.make_async_copy(
            bufs[b], out_hbm.at[pl.ds(0, G_CH)], sems[b]).wait()


_gather_call = functools.partial(
    pl.kernel,
    mesh=plsc.VectorSubcoreMesh(core_axis_name="c", subcore_axis_name="s"),
    compiler_params=pltpu.CompilerParams(use_tc_tiling_on_sc=False),
    out_type=jax.ShapeDtypeStruct((G_ROWS, DP), jnp.float32),
    scratch_types=[
        pltpu.VMEM_SHARED((N_PAD, DP), jnp.float32),
        pltpu.VMEM((G_RPW,), jnp.int32),
        pltpu.VMEM((G_CH, DP), jnp.float32),
        pltpu.VMEM((G_CH, DP), jnp.float32),
        pltpu.SemaphoreType.DMA,
        pltpu.SemaphoreType.DMA,
        pltpu.SemaphoreType.DMA,
    ],
)(_gather_body)


BLK = 1024
NBLK = E_PAD // BLK


def _unpack(packed):
    u = lax.bitcast_convert_type(packed, jnp.uint32)
    lo = lax.bitcast_convert_type(u << 16, jnp.float32)
    hi = lax.bitcast_convert_type(u & jnp.uint32(0xFFFF0000), jnp.float32)
    return lo, hi


def _mlp_body(sent_ref, recv_ref, edge_ref, wq_ref, bq_ref, w1_ref, b1_ref,
              wqe_ref, w1e_ref, w2_ref, b2_ref, w3_ref, b3_ref,
              qe_ref, e_ref):
    s_lo, s_hi = _unpack(sent_ref[...])
    r_lo, r_hi = _unpack(recv_ref[...])
    s_cat = jnp.concatenate((s_lo, s_hi), axis=-1)
    r_cat = jnp.concatenate((r_lo, r_hi), axis=-1)
    eg = edge_ref[...]

    def proj(w, we):
        return (jnp.dot(s_cat, w[:D], preferred_element_type=jnp.float32)
                + jnp.dot(r_cat, w[D:2 * D], preferred_element_type=jnp.float32)
                + jnp.dot(eg, we, preferred_element_type=jnp.float32))

    q = proj(wq_ref[...], wqe_ref[...]) + bq_ref[...]
    a = proj(w1_ref[...], w1e_ref[...]) + b1_ref[...]
    a = jnp.maximum(a, 0.0)
    ones_d = jnp.full((D, D), 1.0 / D, jnp.float32)
    mu = jnp.dot(a, ones_d, preferred_element_type=jnp.float32)
    var = jnp.dot(a * a, ones_d, preferred_element_type=jnp.float32) - mu * mu
    rstd = lax.rsqrt(var + 1e-6)
    z1 = a * rstd - mu * rstd

    b = jnp.dot(z1, w2_ref[...], preferred_element_type=jnp.float32) + b2_ref[...]
    b = jnp.maximum(b, 0.0)
    mu2 = jnp.mean(b, axis=-1, keepdims=True)
    var2 = jnp.mean(b * b, axis=-1, keepdims=True) - mu2 * mu2
    rstd2 = lax.rsqrt(var2 + 1e-6)
    z2 = b * rstd2 - mu2 * rstd2

    logit = jnp.sum(z2 * w3_ref[...], axis=-1, keepdims=True) + b3_ref[...]
    logit = jnp.maximum(logit, 0.0)

    i = pl.program_id(0)
    rows = i * BLK + lax.broadcasted_iota(jnp.int32, (BLK, 1), 0)
    e = jnp.where(rows < E_SELF, jnp.exp(logit), 0.0)

    qe_ref[...] = q * e
    e_ref[...] = e


def _mlp_call(gath, edges_p, wq, bq2, w1, b12, wqe, w1e, w2f, b2f, w3f, b3f):
    const = lambda shape: pl.BlockSpec(shape, lambda i: (0, 0))
    return pl.pallas_call(
        _mlp_body,
        grid=(NBLK,),
        in_specs=[
            pl.BlockSpec((BLK, DP), lambda i: (i, 0)),
            pl.BlockSpec((BLK, DP), lambda i: (i + NBLK, 0)),
            pl.BlockSpec((BLK, DE), lambda i: (i, 0)),
            const((2 * D, F0)),
            const((1, F0)),
            const((2 * D, F0)),
            const((1, F0)),
            const((DE, F0)),
            const((DE, F0)),
            const((F0, F1)),
            const((1, F1)),
            const((1, F1)),
            const((1, 1)),
        ],
        out_specs=[
            pl.BlockSpec((BLK, D), lambda i: (i, 0)),
            pl.BlockSpec((BLK, 1), lambda i: (i, 0)),
        ],
        out_shape=[
            jax.ShapeDtypeStruct((E_PAD, D), jnp.float32),
            jax.ShapeDtypeStruct((E_PAD, 1), jnp.float32),
        ],
    )(gath, gath, edges_p, wq, bq2, w1, b12, wqe, w1e, w2f, b2f, w3f, b3f)


S_RPW = E_PAD // NW
S_CH = 96
S_NCH = S_RPW // S_CH
TILE_N = N_PAD // NS
L = 16


def _scatter_body(zeros_hbm, ridx_hbm, qe_hbm, e_hbm,
                  pq_hbm, pd_hbm,
                  acc_sh, idx_v, t0, t1, e_v, den_v,
                  lsem0, lsem1, ssem0, ssem1):
    c = lax.axis_index("c")
    s = lax.axis_index("s")
    wid = s * NC + c
    base_w = wid * S_RPW

    t_bufs = (t0, t1)
    lsems = (lsem0, lsem1)
    ssems = (ssem0, ssem1)

    def ld_descs(j, b):
        base = base_w + j * S_CH
        return (
            pltpu.make_async_copy(
                ridx_hbm.at[pl.ds(base, S_CH)], idx_v.at[b], lsems[b]),
            pltpu.make_async_copy(
                e_hbm.at[pl.ds(base, S_CH)], e_v.at[b], lsems[b]),
            pltpu.make_async_copy(
                qe_hbm.at[pl.ds(base, S_CH)], t_bufs[b], lsems[b]),
        )

    def scat_desc(b):
        return pltpu.make_async_copy(
            t_bufs[b], acc_sh.at[idx_v.at[b]], ssems[b])

    pltpu.sync_copy(zeros_hbm.at[pl.ds(s * TILE_N, TILE_N)],
                    acc_sh.at[pl.ds(s * TILE_N, TILE_N)])

    def zden(j, _):
        den_v[pl.ds(j * L, L)] = jnp.zeros((L,), jnp.float32)
        return 0

    lax.fori_loop(0, N_PAD // L, zden, 0)
    plsc.subcore_barrier()

    for d in ld_descs(0, 0):
        d.start()

    def body2(j2, _):
        for b in range(2):
            j = 2 * j2 + b
            nb = 1 - b
            for d in ld_descs(j, b):
                d.wait()

            @pl.when(j + 1 < S_NCH)
            def _():
                @pl.when(j >= 1)
                def _():
                    scat_desc(nb).wait()

                for d in ld_descs(j + 1, nb):
                    d.start()

            scat_desc(b).start(add=True)

            def vr(k, _):
                iv = idx_v[b, pl.ds(k * L, L)]
                ev = e_v[b, pl.ds(k * L, L)]
                plsc.addupdate_scatter(den_v, [iv], ev)
                return 0

            lax.fori_loop(0, S_CH // L, vr, 0)
        return 0

    lax.fori_loop(0, S_NCH // 2, body2, 0)
    scat_desc(0).wait()
    scat_desc(1).wait()
    plsc.subcore_barrier()

    pltpu.sync_copy(acc_sh.at[pl.ds(s * TILE_N, TILE_N)],
                    pq_hbm.at[pl.ds(c * N_PAD + s * TILE_N, TILE_N)])
    pltpu.sync_copy(den_v, pd_hbm.at[wid])


_scatter_call = functools.partial(
    pl.kernel,
    mesh=plsc.VectorSubcoreMesh(core_axis_name="c", subcore_axis_name="s"),
    compiler_params=pltpu.CompilerParams(needs_layout_passes=False),
    out_type=[
        jax.ShapeDtypeStruct((NC * N_PAD, D), jnp.float32),
        jax.ShapeDtypeStruct((NW, N_PAD), jnp.float32),
    ],
    scratch_types=[
        pltpu.VMEM_SHARED((N_PAD, D), jnp.float32),
        pltpu.VMEM((2, S_CH), jnp.int32),
        pltpu.VMEM((S_CH, D), jnp.float32),
        pltpu.VMEM((S_CH, D), jnp.float32),
        pltpu.VMEM((2, S_CH), jnp.float32),
        pltpu.VMEM((N_PAD,), jnp.float32),
        pltpu.SemaphoreType.DMA,
        pltpu.SemaphoreType.DMA,
        pltpu.SemaphoreType.DMA,
        pltpu.SemaphoreType.DMA,
    ],
)(_scatter_body)


FBLK = 1280
NFBLK = N_PAD // FBLK


def _fin_body(q0_ref, q1_ref, pd_ref, out_ref):
    agg = q0_ref[...] + q1_ref[...]
    ones = jnp.full((NW, 1), 1.0, jnp.float32)
    dsum = lax.dot_general(pd_ref[...], ones, (((0,), (0,)), ((), ())),
                           preferred_element_type=jnp.float32)
    den = jnp.maximum(dsum, 1e-30)
    x = agg / den
    out_ref[...] = jnp.where(x >= 0.0, x, 0.01 * x)


def _fin_call(pq, pd):
    return pl.pallas_call(
        _fin_body,
        grid=(NFBLK,),
        in_specs=[
            pl.BlockSpec((FBLK, D), lambda i: (i, 0)),
            pl.BlockSpec((FBLK, D), lambda i: (i + NFBLK, 0)),
            pl.BlockSpec((NW, FBLK), lambda i: (0, i)),
        ],
        out_specs=pl.BlockSpec((FBLK, D), lambda i: (i, 0)),
        out_shape=jax.ShapeDtypeStruct((N_PAD, D), jnp.float32),
    )(pq, pq, pd)


def kernel(nodes, edges, senders, receivers, copy_arr, Wq, bq, W1, b1,
           ln1_s, ln1_b, W2, b2, ln2_s, ln2_b, W3, b3):
    ar = jnp.arange(N, dtype=senders.dtype)
    zpad = jnp.zeros((E_PAD - E_SELF,), dtype=senders.dtype)
    send_p = jnp.concatenate((senders, ar, zpad))
    recv_p = jnp.concatenate((receivers, ar, zpad))
    idx_all = jnp.concatenate((send_p, recv_p))
    edges_p = jnp.concatenate(
        (edges, jnp.zeros((E_PAD - E, DE), dtype=edges.dtype)), axis=0)

    nodes_packed = lax.bitcast_convert_type(
        nodes.astype(jnp.bfloat16).reshape(N, DP, 2), jnp.float32)
    nodes_packed = jnp.concatenate(
        (nodes_packed, jnp.zeros((N_PAD - N, DP), jnp.float32)))

    gath = _gather_call(nodes_packed, idx_all)

    def rearr(w):
        return jnp.concatenate((w[0:D:2], w[1:D:2], w[D:2 * D:2],
                                w[D + 1:2 * D:2]))

    w2f = ln1_s[:, None] * W2
    b2f = (b2 + ln1_b @ W2).reshape(1, F1)
    w3f = (ln2_s * W3[:, 0]).reshape(1, F1)
    b3f = (b3 + ln2_b @ W3).reshape(1, 1)
    qe, e = _mlp_call(
        gath, edges_p, rearr(Wq), bq.reshape(1, F0), rearr(W1),
        b1.reshape(1, F0), Wq[2 * D:], W1[2 * D:], w2f, b2f, w3f, b3f)

    zeros_acc = jnp.zeros((N_PAD, D), jnp.float32)
    pq, pd = _scatter_call(zeros_acc, recv_p, qe, e.reshape(E_PAD))

    out = _fin_call(pq, pd)
    return out[:N]


# speedup vs baseline: 6.2488x; 1.1554x over previous
import functools

import jax
import jax.numpy as jnp
from jax import lax
from jax.experimental import pallas as pl
from jax.experimental.pallas import tpu as pltpu
from jax.experimental.pallas import tpu_sc as plsc

N = 10000
E = 320000
D = 128
DE = 16
F0 = 128
F1 = 64

E_SELF = E + N
E_PAD = 331776
NH = 2
E_H = E_PAD // NH
N_PAD = 10240

NC = 2
NS = 16
NW = NC * NS

DP = D // 2
G_ROWS = 2 * E_H
G_RPW = G_ROWS // NW
G_CH = 432
G_NCH = G_RPW // G_CH
G_TILE_N = N_PAD // NS


def _gather_body(nodes_hbm, idx_hbm, out_hbm, tbl_sh, idx_v, rows0, rows1,
                 gsem0, gsem1, wsem0, wsem1):
    c = lax.axis_index("c")
    s = lax.axis_index("s")
    wid = s * NC + c
    base_w = wid * G_RPW
    pltpu.sync_copy(nodes_hbm.at[pl.ds(s * G_TILE_N, G_TILE_N)],
                    tbl_sh.at[pl.ds(s * G_TILE_N, G_TILE_N)])
    pltpu.sync_copy(idx_hbm.at[pl.ds(base_w, G_RPW)], idx_v)
    plsc.subcore_barrier()

    bufs = (rows0, rows1)
    wsems = (wsem0, wsem1)
    gsems = (gsem0, gsem1)

    def g_desc(j, b):
        return pltpu.make_async_copy(
            tbl_sh.at[idx_v.at[pl.ds(j * G_CH, G_CH)]], bufs[b], gsems[b])

    def w_desc(j, b):
        return pltpu.make_async_copy(
            bufs[b], out_hbm.at[pl.ds(base_w + j * G_CH, G_CH)], wsems[b])

    g_desc(0, 0).start()

    def body2(j2, _):
        for b in range(2):
            j = 2 * j2 + b
            nb = 1 - b

            @pl.when(j + 1 < G_NCH)
            def _():
                @pl.when(j >= 1)
                def _():
                    w_desc(j - 1, nb).wait()

                g_desc(j + 1, nb).start()

            g_desc(j, b).wait()
            w_desc(j, b).start()
        return 0

    lax.fori_loop(0, G_NCH // 2, body2, 0)
    w_desc(G_NCH - 2, 0).wait()
    w_desc(G_NCH - 1, 1).wait()


_gather_call = functools.partial(
    pl.kernel,
    mesh=plsc.VectorSubcoreMesh(core_axis_name="c", subcore_axis_name="s"),
    compiler_params=pltpu.CompilerParams(use_tc_tiling_on_sc=False),
    out_type=jax.ShapeDtypeStruct((G_ROWS, DP), jnp.float32),
    scratch_types=[
        pltpu.VMEM_SHARED((N_PAD, DP), jnp.float32),
        pltpu.VMEM((G_RPW,), jnp.int32),
        pltpu.VMEM((G_CH, DP), jnp.float32),
        pltpu.VMEM((G_CH, DP), jnp.float32),
        pltpu.SemaphoreType.DMA,
        pltpu.SemaphoreType.DMA,
        pltpu.SemaphoreType.DMA,
        pltpu.SemaphoreType.DMA,
    ],
)(_gather_body)


BLK = 2048
NBLK = E_H // BLK


def _unpack(packed):
    u = lax.bitcast_convert_type(packed, jnp.uint32)
    lo = lax.bitcast_convert_type(u << 16, jnp.float32)
    hi = lax.bitcast_convert_type(u & jnp.uint32(0xFFFF0000), jnp.float32)
    return lo, hi


def _mlp_body(sent_ref, recv_ref, edge_ref, valid_ref, wq_ref, bq_ref,
              w1_ref, b1_ref, wqe_ref, w1e_ref, w2_ref, b2_ref,
              w3_ref, b3_ref, qe_ref, e_ref):
    SB = BLK // 4
    for t in range(4):
        rows = pl.ds(t * SB, SB)
        s_lo, s_hi = _unpack(sent_ref[rows, :])
        r_lo, r_hi = _unpack(recv_ref[rows, :])
        s_cat = jnp.concatenate((s_lo, s_hi), axis=-1)
        r_cat = jnp.concatenate((r_lo, r_hi), axis=-1)
        eg = edge_ref[rows, :]

        def proj(w, we):
            return (jnp.dot(s_cat, w[:D], preferred_element_type=jnp.float32)
                    + jnp.dot(r_cat, w[D:2 * D],
                              preferred_element_type=jnp.float32)
                    + jnp.dot(eg, we, preferred_element_type=jnp.float32))

        q = proj(wq_ref[...], wqe_ref[...]) + bq_ref[...]
        a = proj(w1_ref[...], w1e_ref[...]) + b1_ref[...]
        a = jnp.maximum(a, 0.0)
        ones_d = jnp.full((D, D), 1.0 / D, jnp.float32)
        mu = jnp.dot(a, ones_d, preferred_element_type=jnp.float32)
        var = jnp.dot(a * a, ones_d,
                      preferred_element_type=jnp.float32) - mu * mu
        rstd = lax.rsqrt(var + 1e-6)
        z1 = a * rstd - mu * rstd

        b = jnp.dot(z1, w2_ref[...],
                    preferred_element_type=jnp.float32) + b2_ref[...]
        b = jnp.maximum(b, 0.0)
        mu2 = jnp.mean(b, axis=-1, keepdims=True)
        var2 = jnp.mean(b * b, axis=-1, keepdims=True) - mu2 * mu2
        rstd2 = lax.rsqrt(var2 + 1e-6)
        z2 = b * rstd2 - mu2 * rstd2

        logit = jnp.sum(z2 * w3_ref[...], axis=-1, keepdims=True) + b3_ref[...]
        logit = jnp.maximum(logit, 0.0)

        e = jnp.exp(logit) * valid_ref[rows, :]

        qe_ref[rows, :] = q * e
        e_ref[rows, :] = e


def _mlp_call(gath, edges_h, valid_h, wq, bq2, w1, b12, wqe, w1e,
              w2f, b2f, w3f, b3f):
    const = lambda shape: pl.BlockSpec(shape, lambda i: (0, 0))
    return pl.pallas_call(
        _mlp_body,
        grid=(NBLK,),
        in_specs=[
            pl.BlockSpec((BLK, DP), lambda i: (i, 0)),
            pl.BlockSpec((BLK, DP), lambda i: (i + NBLK, 0)),
            pl.BlockSpec((BLK, DE), lambda i: (i, 0)),
            pl.BlockSpec((BLK, 1), lambda i: (i, 0)),
            const((2 * D, F0)),
            const((1, F0)),
            const((2 * D, F0)),
            const((1, F0)),
            const((DE, F0)),
            const((DE, F0)),
            const((F0, F1)),
            const((1, F1)),
            const((1, F1)),
            const((1, 1)),
        ],
        out_specs=[
            pl.BlockSpec((BLK, D), lambda i: (i, 0)),
            pl.BlockSpec((BLK, 1), lambda i: (i, 0)),
        ],
        out_shape=[
            jax.ShapeDtypeStruct((E_H, D), jnp.float32),
            jax.ShapeDtypeStruct((E_H, 1), jnp.float32),
        ],
    )(gath, gath, edges_h, valid_h, wq, bq2, w1, b12, wqe, w1e,
      w2f, b2f, w3f, b3f)


S_RPW = E_H // NW
S_CH = 96
S_NCH = S_RPW // S_CH
TILE_N = N_PAD // NS
L = 16


def _scatter_body(zeros_hbm, ridx_hbm, qe_hbm, e_hbm,
                  pq_hbm, pd_hbm,
                  acc_sh, idx_v, t0, t1, e_v, den_v,
                  lsem0, lsem1, ssem0, ssem1):
    c = lax.axis_index("c")
    s = lax.axis_index("s")
    wid = s * NC + c
    base_w = wid * S_RPW

    t_bufs = (t0, t1)
    lsems = (lsem0, lsem1)
    ssems = (ssem0, ssem1)

    def ld_descs(j, b):
        base = base_w + j * S_CH
        return (
            pltpu.make_async_copy(
                ridx_hbm.at[pl.ds(base, S_CH)], idx_v.at[b], lsems[b]),
            pltpu.make_async_copy(
                e_hbm.at[pl.ds(base, S_CH)], e_v.at[b], lsems[b]),
            pltpu.make_async_copy(
                qe_hbm.at[pl.ds(base, S_CH)], t_bufs[b], lsems[b]),
        )

    def scat_desc(b):
        return pltpu.make_async_copy(
            t_bufs[b], acc_sh.at[idx_v.at[b]], ssems[b])

    pltpu.sync_copy(zeros_hbm.at[pl.ds(s * TILE_N, TILE_N)],
                    acc_sh.at[pl.ds(s * TILE_N, TILE_N)])

    def zden(j, _):
        den_v[pl.ds(j * L, L)] = jnp.zeros((L,), jnp.float32)
        return 0

    lax.fori_loop(0, N_PAD // L, zden, 0)
    plsc.subcore_barrier()

    for d in ld_descs(0, 0):
        d.start()

    def body2(j2, _):
        for b in range(2):
            j = 2 * j2 + b
            nb = 1 - b
            for d in ld_descs(j, b):
                d.wait()

            @pl.when(j + 1 < S_NCH)
            def _():
                @pl.when(j >= 1)
                def _():
                    scat_desc(nb).wait()

                for d in ld_descs(j + 1, nb):
                    d.start()

            scat_desc(b).start(add=True)

            def vr(k, _):
                iv = idx_v[b, pl.ds(k * L, L)]
                ev = e_v[b, pl.ds(k * L, L)]
                plsc.addupdate_scatter(den_v, [iv], ev)
                return 0

            lax.fori_loop(0, S_CH // L, vr, 0)
        return 0

    lax.fori_loop(0, S_NCH // 2, body2, 0)
    scat_desc(0).wait()
    scat_desc(1).wait()
    plsc.subcore_barrier()

    pltpu.sync_copy(acc_sh.at[pl.ds(s * TILE_N, TILE_N)],
                    pq_hbm.at[pl.ds(c * N_PAD + s * TILE_N, TILE_N)])
    pltpu.sync_copy(den_v, pd_hbm.at[wid])


_scatter_call = functools.partial(
    pl.kernel,
    mesh=plsc.VectorSubcoreMesh(core_axis_name="c", subcore_axis_name="s"),
    compiler_params=pltpu.CompilerParams(needs_layout_passes=False),
    out_type=[
        jax.ShapeDtypeStruct((NC * N_PAD, D), jnp.float32),
        jax.ShapeDtypeStruct((NW, N_PAD), jnp.float32),
    ],
    scratch_types=[
        pltpu.VMEM_SHARED((N_PAD, D), jnp.float32),
        pltpu.VMEM((2, S_CH), jnp.int32),
        pltpu.VMEM((S_CH, D), jnp.float32),
        pltpu.VMEM((S_CH, D), jnp.float32),
        pltpu.VMEM((2, S_CH), jnp.float32),
        pltpu.VMEM((N_PAD,), jnp.float32),
        pltpu.SemaphoreType.DMA,
        pltpu.SemaphoreType.DMA,
        pltpu.SemaphoreType.DMA,
        pltpu.SemaphoreType.DMA,
    ],
)(_scatter_body)


FBLK = 1280
NFBLK = N_PAD // FBLK


def _fin_body(qa0_ref, qa1_ref, qb0_ref, qb1_ref, pda_ref, pdb_ref, out_ref):
    agg = (qa0_ref[...] + qa1_ref[...]) + (qb0_ref[...] + qb1_ref[...])
    ones = jnp.full((NW, 1), 1.0, jnp.float32)
    dn = (((0,), (0,)), ((), ()))
    dsum = (lax.dot_general(pda_ref[...], ones, dn,
                            preferred_element_type=jnp.float32)
            + lax.dot_general(pdb_ref[...], ones, dn,
                              preferred_element_type=jnp.float32))
    den = jnp.maximum(dsum, 1e-30)
    x = agg / den
    out_ref[...] = jnp.where(x >= 0.0, x, 0.01 * x)


def _fin_call(pq_a, pq_b, pd_a, pd_b):
    return pl.pallas_call(
        _fin_body,
        grid=(NFBLK,),
        in_specs=[
            pl.BlockSpec((FBLK, D), lambda i: (i, 0)),
            pl.BlockSpec((FBLK, D), lambda i: (i + NFBLK, 0)),
            pl.BlockSpec((FBLK, D), lambda i: (i, 0)),
            pl.BlockSpec((FBLK, D), lambda i: (i + NFBLK, 0)),
            pl.BlockSpec((NW, FBLK), lambda i: (0, i)),
            pl.BlockSpec((NW, FBLK), lambda i: (0, i)),
        ],
        out_specs=pl.BlockSpec((FBLK, D), lambda i: (i, 0)),
        out_shape=jax.ShapeDtypeStruct((N_PAD, D), jnp.float32),
    )(pq_a, pq_a, pq_b, pq_b, pd_a, pd_b)


def kernel(nodes, edges, senders, receivers, copy_arr, Wq, bq, W1, b1,
           ln1_s, ln1_b, W2, b2, ln2_s, ln2_b, W3, b3):
    ar = jnp.arange(N, dtype=senders.dtype)
    zpad = jnp.zeros((E_PAD - E_SELF,), dtype=senders.dtype)
    send_p = jnp.concatenate((senders, ar, zpad))
    recv_p = jnp.concatenate((receivers, ar, zpad))
    edges_p = jnp.concatenate(
        (edges, jnp.zeros((E_PAD - E, DE), dtype=edges.dtype)), axis=0)
    valid = (jnp.arange(E_PAD, dtype=jnp.int32) < E_SELF).astype(
        jnp.float32).reshape(E_PAD, 1)

    nodes_packed = lax.bitcast_convert_type(
        nodes.astype(jnp.bfloat16).reshape(N, DP, 2), jnp.float32)
    nodes_packed = jnp.concatenate(
        (nodes_packed, jnp.zeros((N_PAD - N, DP), jnp.float32)))

    def rearr(w):
        return jnp.concatenate((w[0:D:2], w[1:D:2], w[D:2 * D:2],
                                w[D + 1:2 * D:2]))

    w2f = ln1_s[:, None] * W2
    b2f = (b2 + ln1_b @ W2).reshape(1, F1)
    w3f = (ln2_s * W3[:, 0]).reshape(1, F1)
    b3f = (b3 + ln2_b @ W3).reshape(1, 1)
    wqx, w1x = rearr(Wq), rearr(W1)
    bq2, b12 = bq.reshape(1, F0), b1.reshape(1, F0)
    zeros_acc = jnp.zeros((N_PAD, D), jnp.float32)

    parts = []
    for h in range(NH):
        send_h = lax.dynamic_slice(send_p, (h * E_H,), (E_H,))
        recv_h = lax.dynamic_slice(recv_p, (h * E_H,), (E_H,))
        edges_h = lax.dynamic_slice(edges_p, (h * E_H, 0), (E_H, DE))
        valid_h = lax.dynamic_slice(valid, (h * E_H, 0), (E_H, 1))
        idx_h = jnp.concatenate((send_h, recv_h))
        gath_h = _gather_call(nodes_packed, idx_h)
        qe_h, e_h = _mlp_call(
            gath_h, edges_h, valid_h, wqx, bq2, w1x, b12,
            Wq[2 * D:], W1[2 * D:], w2f, b2f, w3f, b3f)
        pq_h, pd_h = _scatter_call(zeros_acc, recv_h, qe_h,
                                   e_h.reshape(E_H))
        parts.append((pq_h, pd_h))

    out = _fin_call(parts[0][0], parts[1][0], parts[0][1], parts[1][1])
    return out[:N]
